# Initial kernel scaffold; baseline (speedup 1.0000x reference)
import jax
import jax.numpy as jnp
from jax.experimental import pallas as pl


def kernel(tuple_values, tuple_rows, tuple_cols, edge_index, msg_src, msg_dst, Wn1, bn1, Wn2, bn2, Wd1, bd1, Wd2, bd2):
    raise NotImplementedError("write your pallas kernel here")


# speedup vs baseline: 1.0008x; 1.0008x over previous
import functools

import jax
import jax.numpy as jnp
from jax import lax
from jax.experimental import pallas as pl
from jax.experimental.pallas import tpu as pltpu

N = 10000
NNZ = 320000
E = 320000
M = 1000000
D = 128


def _mlp_body(x_ref, w1_ref, b1_ref, w2_ref, b2_ref, o_ref):
    x = x_ref[...]
    h = jnp.maximum(
        jnp.dot(x, w1_ref[...], preferred_element_type=jnp.float32) + b1_ref[...],
        0.0,
    )
    o_ref[...] = jnp.maximum(
        jnp.dot(h, w2_ref[...], preferred_element_type=jnp.float32) + b2_ref[...],
        0.0,
    )


def _mlp_pallas(x, W1, b1, W2, b2, blk):
    n = x.shape[0]
    grid = n // blk
    return pl.pallas_call(
        _mlp_body,
        grid=(grid,),
        in_specs=[
            pl.BlockSpec((blk, D), lambda i: (i, 0)),
            pl.BlockSpec((D, D), lambda i: (0, 0)),
            pl.BlockSpec((D,), lambda i: (0,)),
            pl.BlockSpec((D, D), lambda i: (0, 0)),
            pl.BlockSpec((D,), lambda i: (0,)),
        ],
        out_specs=pl.BlockSpec((blk, D), lambda i: (i, 0)),
        out_shape=jax.ShapeDtypeStruct((n, D), jnp.float32),
    )(x, W1, b1, W2, b2)


def kernel(tuple_values, tuple_rows, tuple_cols, edge_index, msg_src, msg_dst,
           Wn1, bn1, Wn2, bn2, Wd1, bd1, Wd2, bd2):
    tuple_rows = tuple_rows.astype(jnp.int32)
    tuple_cols = tuple_cols.astype(jnp.int32)
    edge_src = edge_index[0].astype(jnp.int32)
    edge_dst = edge_index[1].astype(jnp.int32)
    msg_src = msg_src.astype(jnp.int32)
    msg_dst = msg_dst.astype(jnp.int32)

    tX = _mlp_pallas(tuple_values, Wn1, bn1, Wn2, bn2, blk=1600)
    ret1 = jnp.zeros((NNZ, D), jnp.float32).at[msg_dst].add(tX[msg_src])

    nodex = jax.ops.segment_max(tuple_values, tuple_rows, num_segments=N)
    nodex = jnp.where(jnp.isfinite(nodex), nodex, 0.0)
    nodex = _mlp_pallas(nodex, Wd1, bd1, Wd2, bd2, blk=1000)
    nodex_mp = jnp.zeros_like(nodex).at[edge_dst].add(nodex[edge_src])
    ret2 = nodex_mp[tuple_cols]
    return ret2 + ret1
